# baseline (speedup 1.0000x reference)
import functools

import jax
import jax.numpy as jnp
from jax import lax
from jax.experimental import pallas as pl
from jax.experimental.pallas import tpu as pltpu
from jax.experimental.pallas import tpu_sc as plsc

N = 50000
H = 64
HC = 32
G = 16
OUT = 128
L = 4

NC = 2
NS = 16
B = 128
E_PAD = 819200
NB = E_PAD // B
S_ROWS = 51200
RT = S_ROWS // NS

DUMMY_DST = N

ROW_BLK = 2000


def _fill_f32(ref, rows, cols, val):
    v = jnp.full((16,), val, jnp.float32)

    def body(i, _):
        for c0 in range(0, cols, 16):
            ref[i, c0:c0 + 16] = v
        return 0

    lax.fori_loop(0, rows, body, 0)


def _sc_degree(col2):
    nbt = NB // (NC * NS)
    mesh = plsc.VectorSubcoreMesh(core_axis_name="c", subcore_axis_name="s")

    @functools.partial(
        pl.kernel,
        mesh=mesh,
        compiler_params=pltpu.CompilerParams(use_tc_tiling_on_sc=False),
        out_type=[
            jax.ShapeDtypeStruct((S_ROWS, 16), jnp.float32),
            jax.ShapeDtypeStruct((S_ROWS, 16), jnp.float32),
        ],
        scratch_types=[
            pltpu.VMEM((nbt, B), jnp.int32),
            pltpu.VMEM((B, 16), jnp.float32),
            pltpu.VMEM((B, 16), jnp.float32),
            pltpu.VMEM_SHARED((S_ROWS, 16), jnp.float32),
        ],
    )
    def k(col_hbm, d0_hbm, d1_hbm, cidx_v, ones_v, zero_v, deg_sh):
        cid = lax.axis_index("c")
        sid = lax.axis_index("s")
        wid = sid * NC + cid

        _fill_f32(ones_v, B, 16, 1.0)
        _fill_f32(zero_v, B, 16, 0.0)

        def zbody(j, _):
            pltpu.sync_copy(zero_v, deg_sh.at[pl.ds(sid * RT + j * B, B)])
            return 0
        lax.fori_loop(0, RT // B, zbody, 0)

        pltpu.sync_copy(col_hbm.at[pl.ds(wid * nbt, nbt)], cidx_v)

        plsc.subcore_barrier()

        def sbody(g, _):
            pltpu.sync_copy(ones_v, deg_sh.at[cidx_v.at[g]], add=True)
            return 0
        lax.fori_loop(0, nbt, sbody, 0)

        plsc.subcore_barrier()

        @pl.when(cid == 0)
        def _():
            pltpu.sync_copy(deg_sh.at[pl.ds(sid * RT, RT)],
                            d0_hbm.at[pl.ds(sid * RT, RT)])

        @pl.when(cid == 1)
        def _():
            pltpu.sync_copy(deg_sh.at[pl.ds(sid * RT, RT)],
                            d1_hbm.at[pl.ds(sid * RT, RT)])

    return k(col2)


BL = 64


def _sc_layer(rowL, colL, hws_a, hws_b):
    nbt = E_PAD // BL // NS
    mesh = plsc.VectorSubcoreMesh(core_axis_name="c", subcore_axis_name="s")

    QB = 40
    NBUF = 10
    LOOK = 5

    @functools.partial(
        pl.kernel,
        mesh=mesh,
        compiler_params=pltpu.CompilerParams(use_tc_tiling_on_sc=False),
        out_type=[
            jax.ShapeDtypeStruct((S_ROWS, HC), jnp.float32),
            jax.ShapeDtypeStruct((S_ROWS, HC), jnp.float32),
        ],
        scratch_types=[
            pltpu.VMEM((QB, BL), jnp.int32),
            pltpu.VMEM((QB, BL), jnp.int32),
            pltpu.VMEM((NBUF * BL, HC), jnp.float32),
            pltpu.VMEM_SHARED((S_ROWS, HC), jnp.float32),
        ] + [pltpu.SemaphoreType.DMA] * NBUF,
    )
    def k(row_hbm, col_hbm, ha_hbm, hb_hbm, s0_hbm, s1_hbm,
          ridx_v, cidx_v, rows_v, s_sh, *sems):
        cid = lax.axis_index("c")
        sid = lax.axis_index("s")

        _fill_f32(rows_v, 2 * BL, HC, 0.0)

        def zbody(j, _):
            pltpu.sync_copy(rows_v.at[pl.ds(0, 2 * BL)],
                            s_sh.at[pl.ds(sid * RT + j * 2 * BL, 2 * BL)])
            return 0
        lax.fori_loop(0, RT // (2 * BL), zbody, 0)

        plsc.subcore_barrier()

        def run(tab_hbm):
            def buf(b):
                return rows_v.at[pl.ds(b * BL, BL)]

            def gather(j, b):
                pltpu.async_copy(tab_hbm.at[ridx_v.at[j]], buf(b), sems[b])

            def scatter(j, b):
                pltpu.async_copy(buf(b), s_sh.at[cidx_v.at[j]],
                                 sems[b], add=True)

            def wait(b):
                pltpu.make_async_copy(buf(b), s_sh.at[cidx_v.at[0]],
                                      sems[b]).wait()

            def slot(q, _):
                base = sid * nbt + q * QB
                pltpu.sync_copy(row_hbm.at[pl.ds(base, QB)], ridx_v)
                pltpu.sync_copy(col_hbm.at[pl.ds(base, QB)], cidx_v)
                for j in range(LOOK):
                    gather(j, j % NBUF)
                for j in range(QB):
                    b = j % NBUF
                    wait(b)
                    scatter(j, b)
                    jn = j + LOOK
                    if jn < QB:
                        b2 = jn % NBUF
                        if jn >= NBUF:
                            wait(b2)
                        gather(jn, b2)
                for j in range(QB - NBUF, QB):
                    wait(j % NBUF)
                return 0
            lax.fori_loop(0, nbt // QB, slot, 0)

        @pl.when(cid == 0)
        def _():
            run(ha_hbm)

        @pl.when(cid == 1)
        def _():
            run(hb_hbm)

        plsc.subcore_barrier()

        @pl.when(cid == 0)
        def _():
            pltpu.sync_copy(s_sh.at[pl.ds(sid * RT, RT)],
                            s0_hbm.at[pl.ds(sid * RT, RT)])

        @pl.when(cid == 1)
        def _():
            pltpu.sync_copy(s_sh.at[pl.ds(sid * RT, RT)],
                            s1_hbm.at[pl.ds(sid * RT, RT)])

    return k(rowL, colL, hws_a, hws_b)


def _tc_pre_body(x_ref, d0_ref, d1_ref, we_ref, be_ref, w0_ref,
                 h_ref, dis_ref, ha_ref, hb_ref):
    xb = x_ref[...]
    h = jnp.maximum(
        jnp.dot(xb, we_ref[...], preferred_element_type=jnp.float32)
        + be_ref[...], 0.0)
    deg = d0_ref[:, 0:1] + d1_ref[:, 0:1] + 1.0
    dis = lax.rsqrt(deg)
    hws = dis * jnp.dot(h, w0_ref[...], preferred_element_type=jnp.float32)
    h_ref[...] = h
    dis_ref[...] = dis
    ha_ref[...] = hws[:, :HC]
    hb_ref[...] = hws[:, HC:]


def _tc_pre(x, d0, d1, We, be, W0):
    grid = (N // ROW_BLK,)
    return pl.pallas_call(
        _tc_pre_body,
        grid=grid,
        in_specs=[
            pl.BlockSpec((ROW_BLK, 9), lambda i: (i, 0)),
            pl.BlockSpec((ROW_BLK, 16), lambda i: (i, 0)),
            pl.BlockSpec((ROW_BLK, 16), lambda i: (i, 0)),
            pl.BlockSpec((9, H), lambda i: (0, 0)),
            pl.BlockSpec((1, H), lambda i: (0, 0)),
            pl.BlockSpec((H, H), lambda i: (0, 0)),
        ],
        out_specs=[
            pl.BlockSpec((ROW_BLK, H), lambda i: (i, 0)),
            pl.BlockSpec((ROW_BLK, 1), lambda i: (i, 0)),
            pl.BlockSpec((ROW_BLK, HC), lambda i: (i, 0)),
            pl.BlockSpec((ROW_BLK, HC), lambda i: (i, 0)),
        ],
        out_shape=[
            jax.ShapeDtypeStruct((N, H), jnp.float32),
            jax.ShapeDtypeStruct((N, 1), jnp.float32),
            jax.ShapeDtypeStruct((N, HC), jnp.float32),
            jax.ShapeDtypeStruct((N, HC), jnp.float32),
        ],
    )(x, d0, d1, We, be, W0)


def _layer_update(h_ref, dis_ref, s0_ref, s1_ref, ha_ref, hb_ref,
                  b_ref, g_ref, bt_ref):
    S = jnp.concatenate([s0_ref[...], s1_ref[...]], axis=1)
    hws = jnp.concatenate([ha_ref[...], hb_ref[...]], axis=1)
    dis = dis_ref[...]
    agg = dis * (S + hws) + b_ref[...]
    mu = jnp.mean(agg, axis=1, keepdims=True)
    diff = agg - mu
    var = jnp.mean(diff * diff, axis=1, keepdims=True)
    hn = diff * lax.rsqrt(var + 1e-5) * g_ref[...] + bt_ref[...]
    return h_ref[...] + jnp.maximum(hn, 0.0), dis


def _tc_layer_body(h_ref, dis_ref, s0_ref, s1_ref, ha_ref, hb_ref,
                   b_ref, g_ref, bt_ref, wn_ref,
                   ho_ref, hao_ref, hbo_ref):
    h_new, dis = _layer_update(h_ref, dis_ref, s0_ref, s1_ref, ha_ref,
                               hb_ref, b_ref, g_ref, bt_ref)
    ho_ref[...] = h_new
    hws = dis * jnp.dot(h_new, wn_ref[...], preferred_element_type=jnp.float32)
    hao_ref[...] = hws[:, :HC]
    hbo_ref[...] = hws[:, HC:]


def _tc_layer(h, dis, s0, s1, ha, hb, b, g, bt, Wn):
    grid = (N // ROW_BLK,)
    rb = lambda i: (i, 0)
    z = lambda i: (0, 0)
    return pl.pallas_call(
        _tc_layer_body,
        grid=grid,
        in_specs=[
            pl.BlockSpec((ROW_BLK, H), rb),
            pl.BlockSpec((ROW_BLK, 1), rb),
            pl.BlockSpec((ROW_BLK, HC), rb),
            pl.BlockSpec((ROW_BLK, HC), rb),
            pl.BlockSpec((ROW_BLK, HC), rb),
            pl.BlockSpec((ROW_BLK, HC), rb),
            pl.BlockSpec((1, H), z),
            pl.BlockSpec((1, H), z),
            pl.BlockSpec((1, H), z),
            pl.BlockSpec((H, H), z),
        ],
        out_specs=[
            pl.BlockSpec((ROW_BLK, H), rb),
            pl.BlockSpec((ROW_BLK, HC), rb),
            pl.BlockSpec((ROW_BLK, HC), rb),
        ],
        out_shape=[
            jax.ShapeDtypeStruct((N, H), jnp.float32),
            jax.ShapeDtypeStruct((N, HC), jnp.float32),
            jax.ShapeDtypeStruct((N, HC), jnp.float32),
        ],
    )(h, dis, s0, s1, ha, hb, b, g, bt, Wn)


def _tc_final_body(h_ref, dis_ref, s0_ref, s1_ref, ha_ref, hb_ref,
                   b_ref, g_ref, bt_ref, batch_ref,
                   wo1_ref, bo1_ref, wo2_ref, bo2_ref,
                   out_ref, pooled_ref, cnt_ref):
    step = pl.program_id(0)
    nsteps = pl.num_programs(0)
    h_new, _ = _layer_update(h_ref, dis_ref, s0_ref, s1_ref, ha_ref,
                             hb_ref, b_ref, g_ref, bt_ref)
    bb = batch_ref[...]
    oh = (bb == lax.broadcasted_iota(jnp.int32, (1, G), 1)).astype(jnp.float32)
    dn = (((0,), (0,)), ((), ()))
    psum = lax.dot_general(oh, h_new, dn, preferred_element_type=jnp.float32)
    csum = lax.dot_general(oh, jnp.ones((oh.shape[0], 1), jnp.float32), dn,
                           preferred_element_type=jnp.float32)

    @pl.when(step == 0)
    def _():
        pooled_ref[...] = psum
        cnt_ref[...] = csum

    @pl.when(step > 0)
    def _():
        pooled_ref[...] += psum
        cnt_ref[...] += csum

    @pl.when(step == nsteps - 1)
    def _():
        pooled = pooled_ref[...] / jnp.maximum(cnt_ref[...], 1.0)
        t = jnp.maximum(
            jnp.dot(pooled, wo1_ref[...], preferred_element_type=jnp.float32)
            + bo1_ref[...], 0.0)
        out_ref[...] = (
            jnp.dot(t, wo2_ref[...], preferred_element_type=jnp.float32)
            + bo2_ref[...])


def _tc_final(h, dis, s0, s1, ha, hb, b, g, bt, batch2,
              Wo1, bo1, Wo2, bo2):
    grid = (N // ROW_BLK,)
    rb = lambda i: (i, 0)
    z = lambda i: (0, 0)
    return pl.pallas_call(
        _tc_final_body,
        grid=grid,
        in_specs=[
            pl.BlockSpec((ROW_BLK, H), rb),
            pl.BlockSpec((ROW_BLK, 1), rb),
            pl.BlockSpec((ROW_BLK, HC), rb),
            pl.BlockSpec((ROW_BLK, HC), rb),
            pl.BlockSpec((ROW_BLK, HC), rb),
            pl.BlockSpec((ROW_BLK, HC), rb),
            pl.BlockSpec((1, H), z),
            pl.BlockSpec((1, H), z),
            pl.BlockSpec((1, H), z),
            pl.BlockSpec((ROW_BLK, 1), rb),
            pl.BlockSpec((H, OUT), z),
            pl.BlockSpec((1, OUT), z),
            pl.BlockSpec((OUT, OUT), z),
            pl.BlockSpec((1, OUT), z),
        ],
        out_specs=pl.BlockSpec((G, OUT), z),
        out_shape=jax.ShapeDtypeStruct((G, OUT), jnp.float32),
        scratch_shapes=[
            pltpu.VMEM((G, H), jnp.float32),
            pltpu.VMEM((G, 1), jnp.float32),
        ],
    )(h, dis, s0, s1, ha, hb, b, g, bt, batch2, Wo1, bo1, Wo2, bo2)


def kernel(x, edge_index, batch, W_embed, b_embed, Ws, bs, gammas, betas,
           W_o1, b_o1, W_o2, b_o2):
    E = edge_index.shape[1]
    npad = E_PAD - E
    row_flat = jnp.concatenate(
        [edge_index[0], jnp.zeros((npad,), jnp.int32)])
    col_flat = jnp.concatenate(
        [edge_index[1], jnp.full((npad,), DUMMY_DST, jnp.int32)])
    col = col_flat.reshape(NB, B)
    rowL = row_flat.reshape(E_PAD // BL, BL)
    colL = col_flat.reshape(E_PAD // BL, BL)

    d0, d1 = _sc_degree(col)
    h, dis, ha, hb = _tc_pre(x, d0, d1, W_embed,
                             b_embed.reshape(1, H), Ws[0])
    for l in range(L):
        s0, s1 = _sc_layer(rowL, colL, ha, hb)
        if l < L - 1:
            h, ha, hb = _tc_layer(h, dis, s0, s1, ha, hb,
                                  bs[l].reshape(1, H),
                                  gammas[l].reshape(1, H),
                                  betas[l].reshape(1, H), Ws[l + 1])
        else:
            out = _tc_final(h, dis, s0, s1, ha, hb,
                            bs[l].reshape(1, H),
                            gammas[l].reshape(1, H),
                            betas[l].reshape(1, H),
                            batch.reshape(N, 1),
                            W_o1, b_o1.reshape(1, OUT),
                            W_o2, b_o2.reshape(1, OUT))
    return out


# speedup vs baseline: 1.3522x; 1.3522x over previous
import functools

import jax
import jax.numpy as jnp
from jax import lax
from jax.experimental import pallas as pl
from jax.experimental.pallas import tpu as pltpu
from jax.experimental.pallas import tpu_sc as plsc

N = 50000
H = 64
HC = 32
G = 16
OUT = 128
L = 4

NC = 2
NS = 16
B = 128
E_PAD = 819200
NB = E_PAD // B
S_ROWS = 51200
RT = S_ROWS // NS

DUMMY_DST = N

ROW_BLK = 2000


def _fill_f32(ref, rows, cols, val):
    v = jnp.full((16,), val, jnp.float32)

    def body(i, _):
        for c0 in range(0, cols, 16):
            ref[i, c0:c0 + 16] = v
        return 0

    lax.fori_loop(0, rows, body, 0)


def _sc_degree(col2):
    nbt = NB // (NC * NS)
    mesh = plsc.VectorSubcoreMesh(core_axis_name="c", subcore_axis_name="s")

    @functools.partial(
        pl.kernel,
        mesh=mesh,
        compiler_params=pltpu.CompilerParams(use_tc_tiling_on_sc=False),
        out_type=[
            jax.ShapeDtypeStruct((S_ROWS, 32), jnp.float32),
            jax.ShapeDtypeStruct((S_ROWS, 32), jnp.float32),
        ],
        scratch_types=[
            pltpu.VMEM((nbt // 2, B), jnp.int32),
            pltpu.VMEM((B, 32), jnp.float32),
            pltpu.VMEM((B, 32), jnp.float32),
            pltpu.VMEM_SHARED((S_ROWS, 32), jnp.float32),
        ],
    )
    def k(col_hbm, d0_hbm, d1_hbm, cidx_v, ones_v, zero_v, deg_sh):
        cid = lax.axis_index("c")
        sid = lax.axis_index("s")
        wid = sid * NC + cid

        _fill_f32(ones_v, B, 32, 1.0)
        _fill_f32(zero_v, B, 32, 0.0)

        def zbody(j, _):
            pltpu.sync_copy(zero_v, deg_sh.at[pl.ds(sid * RT + j * B, B)])
            return 0
        lax.fori_loop(0, RT // B, zbody, 0)

        plsc.subcore_barrier()

        for half in range(2):
            pltpu.sync_copy(
                col_hbm.at[pl.ds(wid * nbt + half * (nbt // 2), nbt // 2)],
                cidx_v)

            def sbody(g, _):
                pltpu.sync_copy(ones_v, deg_sh.at[cidx_v.at[g]], add=True)
                return 0
            lax.fori_loop(0, nbt // 2, sbody, 0)

        plsc.subcore_barrier()

        @pl.when(cid == 0)
        def _():
            pltpu.sync_copy(deg_sh.at[pl.ds(sid * RT, RT)],
                            d0_hbm.at[pl.ds(sid * RT, RT)])

        @pl.when(cid == 1)
        def _():
            pltpu.sync_copy(deg_sh.at[pl.ds(sid * RT, RT)],
                            d1_hbm.at[pl.ds(sid * RT, RT)])

    return k(col2)


BL = 128


def _sc_layer(rowL, colL, hws_a, hws_b):
    nbt = E_PAD // BL // NS
    mesh = plsc.VectorSubcoreMesh(core_axis_name="c", subcore_axis_name="s")

    QB = 25
    NBUF = 5
    LOOK = 3

    @functools.partial(
        pl.kernel,
        mesh=mesh,
        compiler_params=pltpu.CompilerParams(use_tc_tiling_on_sc=False),
        out_type=[
            jax.ShapeDtypeStruct((S_ROWS, HC), jnp.float32),
            jax.ShapeDtypeStruct((S_ROWS, HC), jnp.float32),
        ],
        scratch_types=[
            pltpu.VMEM((QB, BL), jnp.int32),
            pltpu.VMEM((QB, BL), jnp.int32),
            pltpu.VMEM((NBUF * BL, HC), jnp.float32),
            pltpu.VMEM_SHARED((S_ROWS, HC), jnp.float32),
        ] + [pltpu.SemaphoreType.DMA] * NBUF,
    )
    def k(row_hbm, col_hbm, ha_hbm, hb_hbm, s0_hbm, s1_hbm,
          ridx_v, cidx_v, rows_v, s_sh, *sems):
        cid = lax.axis_index("c")
        sid = lax.axis_index("s")

        _fill_f32(rows_v, 2 * BL, HC, 0.0)

        def zbody(j, _):
            pltpu.sync_copy(rows_v.at[pl.ds(0, 2 * BL)],
                            s_sh.at[pl.ds(sid * RT + j * 2 * BL, 2 * BL)])
            return 0
        lax.fori_loop(0, RT // (2 * BL), zbody, 0)

        plsc.subcore_barrier()

        def run(tab_hbm):
            def buf(b):
                return rows_v.at[pl.ds(b * BL, BL)]

            def gather(j, b):
                pltpu.async_copy(tab_hbm.at[ridx_v.at[j]], buf(b), sems[b])

            def scatter(j, b):
                pltpu.async_copy(buf(b), s_sh.at[cidx_v.at[j]],
                                 sems[b], add=True)

            def wait(b):
                pltpu.make_async_copy(buf(b), s_sh.at[cidx_v.at[0]],
                                      sems[b]).wait()

            def slot(q, _):
                base = sid * nbt + q * QB
                pltpu.sync_copy(row_hbm.at[pl.ds(base, QB)], ridx_v)
                pltpu.sync_copy(col_hbm.at[pl.ds(base, QB)], cidx_v)
                for j in range(LOOK):
                    gather(j, j % NBUF)
                for j in range(QB):
                    b = j % NBUF
                    wait(b)
                    scatter(j, b)
                    jn = j + LOOK
                    if jn < QB:
                        b2 = jn % NBUF
                        if jn >= NBUF:
                            wait(b2)
                        gather(jn, b2)
                for j in range(QB - NBUF, QB):
                    wait(j % NBUF)
                return 0
            lax.fori_loop(0, nbt // QB, slot, 0)

        @pl.when(cid == 0)
        def _():
            run(ha_hbm)

        @pl.when(cid == 1)
        def _():
            run(hb_hbm)

        plsc.subcore_barrier()

        @pl.when(cid == 0)
        def _():
            pltpu.sync_copy(s_sh.at[pl.ds(sid * RT, RT)],
                            s0_hbm.at[pl.ds(sid * RT, RT)])

        @pl.when(cid == 1)
        def _():
            pltpu.sync_copy(s_sh.at[pl.ds(sid * RT, RT)],
                            s1_hbm.at[pl.ds(sid * RT, RT)])

    return k(rowL, colL, hws_a, hws_b)


RBP = 1280
NP = S_ROWS * HC // 128


def _rb(i):
    return (i, 0)


def _z(i):
    return (0, 0)


def _tc_pre_body(xp_ref, d0_ref, d1_ref, wea_ref, web_ref, bea_ref, beb_ref,
                 w_refs, ha_ref, hb_ref, dis_ref, hwa_ref, hwb_ref):
    xp = xp_ref[...]
    h_a = jnp.maximum(
        jnp.dot(xp, wea_ref[...], preferred_element_type=jnp.float32)
        + bea_ref[...], 0.0)
    h_b = jnp.maximum(
        jnp.dot(xp, web_ref[...], preferred_element_type=jnp.float32)
        + beb_ref[...], 0.0)
    dis = lax.rsqrt(d0_ref[...] + d1_ref[...] + 1.0)
    waa, wba, wab, wbb = w_refs
    hws_a = dis * (jnp.dot(h_a, waa[...], preferred_element_type=jnp.float32)
                   + jnp.dot(h_b, wba[...], preferred_element_type=jnp.float32))
    hws_b = dis * (jnp.dot(h_a, wab[...], preferred_element_type=jnp.float32)
                   + jnp.dot(h_b, wbb[...], preferred_element_type=jnp.float32))
    ha_ref[...] = h_a
    hb_ref[...] = h_b
    dis_ref[...] = dis
    hwa_ref[...] = hws_a
    hwb_ref[...] = hws_b


def _tc_pre(xp, d0, d1, wea, web, bea, beb, wbd):
    grid = (NP // RBP,)
    return pl.pallas_call(
        lambda xr, a, b, c, d, e, f, w0, w1, w2, w3, *outs: _tc_pre_body(
            xr, a, b, c, d, e, f, (w0, w1, w2, w3), *outs),
        grid=grid,
        in_specs=[
            pl.BlockSpec((RBP, 36), _rb),
            pl.BlockSpec((RBP, 128), _rb),
            pl.BlockSpec((RBP, 128), _rb),
            pl.BlockSpec((36, 128), _z),
            pl.BlockSpec((36, 128), _z),
            pl.BlockSpec((1, 128), _z),
            pl.BlockSpec((1, 128), _z),
        ] + [pl.BlockSpec((128, 128), _z)] * 4,
        out_specs=[pl.BlockSpec((RBP, 128), _rb)] * 5,
        out_shape=[jax.ShapeDtypeStruct((NP, 128), jnp.float32)] * 5,
    )(xp, d0, d1, wea, web, bea, beb, *wbd)


def _layer_update(h_a, h_b, dis, s0, s1, hwa, hwb,
                  ba, bb, ga, gb, bta, btb, m_ref):
    agg_a = dis * (s0 + hwa) + ba
    agg_b = dis * (s1 + hwb) + bb
    m = m_ref[...]
    mu = jnp.dot(agg_a + agg_b, m, preferred_element_type=jnp.float32) / 64.0
    da = agg_a - mu
    db = agg_b - mu
    var = jnp.dot(da * da + db * db, m,
                  preferred_element_type=jnp.float32) / 64.0
    rstd = lax.rsqrt(var + 1e-5)
    hn_a = da * rstd * ga + bta
    hn_b = db * rstd * gb + btb
    return h_a + jnp.maximum(hn_a, 0.0), h_b + jnp.maximum(hn_b, 0.0)


def _tc_layer_body(ha_ref, hb_ref, dis_ref, s0_ref, s1_ref, hwa_ref, hwb_ref,
                   ba_ref, bb_ref, ga_ref, gb_ref, bta_ref, btb_ref, m_ref,
                   w_refs, hao_ref, hbo_ref, hwao_ref, hwbo_ref):
    dis = dis_ref[...]
    h_a, h_b = _layer_update(
        ha_ref[...], hb_ref[...], dis, s0_ref[...], s1_ref[...],
        hwa_ref[...], hwb_ref[...], ba_ref[...], bb_ref[...],
        ga_ref[...], gb_ref[...], bta_ref[...], btb_ref[...], m_ref)
    waa, wba, wab, wbb = w_refs
    hao_ref[...] = h_a
    hbo_ref[...] = h_b
    hwao_ref[...] = dis * (
        jnp.dot(h_a, waa[...], preferred_element_type=jnp.float32)
        + jnp.dot(h_b, wba[...], preferred_element_type=jnp.float32))
    hwbo_ref[...] = dis * (
        jnp.dot(h_a, wab[...], preferred_element_type=jnp.float32)
        + jnp.dot(h_b, wbb[...], preferred_element_type=jnp.float32))


def _tc_layer(h_a, h_b, dis, s0, s1, hwa, hwb, consts, m, wbd):
    grid = (NP // RBP,)
    return pl.pallas_call(
        lambda a, b, c, d, e, f, g2, c1, c2, c3, c4, c5, c6, mm,
               w0, w1, w2, w3, *outs: _tc_layer_body(
            a, b, c, d, e, f, g2, c1, c2, c3, c4, c5, c6, mm,
            (w0, w1, w2, w3), *outs),
        grid=grid,
        in_specs=[pl.BlockSpec((RBP, 128), _rb)] * 7
        + [pl.BlockSpec((1, 128), _z)] * 6
        + [pl.BlockSpec((128, 128), _z)]
        + [pl.BlockSpec((128, 128), _z)] * 4,
        out_specs=[pl.BlockSpec((RBP, 128), _rb)] * 4,
        out_shape=[jax.ShapeDtypeStruct((NP, 128), jnp.float32)] * 4,
    )(h_a, h_b, dis, s0, s1, hwa, hwb, *consts, m, *wbd)


def _tc_final_body(ha_ref, hb_ref, dis_ref, s0_ref, s1_ref, hwa_ref, hwb_ref,
                   ba_ref, bb_ref, ga_ref, gb_ref, bta_ref, btb_ref, m_ref,
                   bp_ref, wo1_ref, bo1_ref, wo2_ref, bo2_ref,
                   out_ref, pa_ref, pb_ref, cnt_ref):
    step = pl.program_id(0)
    nsteps = pl.num_programs(0)
    h_a, h_b = _layer_update(
        ha_ref[...], hb_ref[...], dis_ref[...], s0_ref[...], s1_ref[...],
        hwa_ref[...], hwb_ref[...], ba_ref[...], bb_ref[...],
        ga_ref[...], gb_ref[...], bta_ref[...], btb_ref[...], m_ref)
    bp = bp_ref[...]
    dn = (((0,), (0,)), ((), ()))
    ones_col = jnp.ones((RBP, 1), jnp.float32)
    pa = jnp.zeros((G, HC), jnp.float32)
    pb = jnp.zeros((G, HC), jnp.float32)
    cnt = jnp.zeros((G, 1), jnp.float32)
    iota_g = lax.broadcasted_iota(jnp.int32, (1, G), 1)
    for q in range(4):
        bq = bp[:, q:q + 1]
        oh = (bq == iota_g).astype(jnp.float32)
        valid = (bq < G) & (bq >= 0)
        ya = lax.dot_general(
            oh, jnp.where(valid, h_a[:, 32 * q:32 * q + 32], 0.0), dn,
            preferred_element_type=jnp.float32)
        yb = lax.dot_general(
            oh, jnp.where(valid, h_b[:, 32 * q:32 * q + 32], 0.0), dn,
            preferred_element_type=jnp.float32)
        pa = pa + ya
        pb = pb + yb
        cnt = cnt + lax.dot_general(oh, ones_col, dn,
                                    preferred_element_type=jnp.float32)

    @pl.when(step == 0)
    def _():
        pa_ref[...] = pa
        pb_ref[...] = pb
        cnt_ref[...] = cnt

    @pl.when(step > 0)
    def _():
        pa_ref[...] += pa
        pb_ref[...] += pb
        cnt_ref[...] += cnt

    @pl.when(step == nsteps - 1)
    def _():
        c = jnp.maximum(cnt_ref[...], 1.0)
        pooled = jnp.concatenate([pa_ref[...], pb_ref[...]], axis=1) / c
        t = jnp.maximum(
            jnp.dot(pooled, wo1_ref[...], preferred_element_type=jnp.float32)
            + bo1_ref[...], 0.0)
        out_ref[...] = (
            jnp.dot(t, wo2_ref[...], preferred_element_type=jnp.float32)
            + bo2_ref[...])


def _tc_final(h_a, h_b, dis, s0, s1, hwa, hwb, consts, m, bp,
              Wo1, bo1, Wo2, bo2):
    grid = (NP // RBP,)
    return pl.pallas_call(
        _tc_final_body,
        grid=grid,
        in_specs=[pl.BlockSpec((RBP, 128), _rb)] * 7
        + [pl.BlockSpec((1, 128), _z)] * 6
        + [pl.BlockSpec((128, 128), _z)]
        + [pl.BlockSpec((RBP, 4), _rb)]
        + [pl.BlockSpec((H, OUT), _z), pl.BlockSpec((1, OUT), _z),
           pl.BlockSpec((OUT, OUT), _z), pl.BlockSpec((1, OUT), _z)],
        out_specs=pl.BlockSpec((G, OUT), _z),
        out_shape=jax.ShapeDtypeStruct((G, OUT), jnp.float32),
        scratch_shapes=[
            pltpu.VMEM((G, HC), jnp.float32),
            pltpu.VMEM((G, HC), jnp.float32),
            pltpu.VMEM((G, 1), jnp.float32),
        ],
    )(h_a, h_b, dis, s0, s1, hwa, hwb, *consts, m, bp, Wo1, bo1, Wo2, bo2)


def _bd(w):
    return jnp.kron(jnp.eye(4, dtype=w.dtype), w)


def _lane4(v):
    return jnp.tile(v, 4).reshape(1, 128)


def kernel(x, edge_index, batch, W_embed, b_embed, Ws, bs, gammas, betas,
           W_o1, b_o1, W_o2, b_o2):
    E = edge_index.shape[1]
    npad = E_PAD - E
    row_flat = jnp.concatenate(
        [edge_index[0], jnp.zeros((npad,), jnp.int32)])
    col_flat = jnp.concatenate(
        [edge_index[1], jnp.full((npad,), DUMMY_DST, jnp.int32)])
    col = col_flat.reshape(NB, B)
    rowL = row_flat.reshape(E_PAD // BL, BL)
    colL = col_flat.reshape(E_PAD // BL, BL)

    nrp = S_ROWS - N
    xp = jnp.concatenate(
        [x, jnp.zeros((nrp, x.shape[1]), x.dtype)]).reshape(NP, 36)
    bp = jnp.concatenate(
        [batch, jnp.full((nrp,), G, batch.dtype)]).reshape(NP, 4)

    lane = jnp.arange(128) // 32
    m = (lane[:, None] == lane[None, :]).astype(jnp.float32)

    wea = jnp.kron(jnp.eye(4, dtype=x.dtype), W_embed[:, :HC])
    web = jnp.kron(jnp.eye(4, dtype=x.dtype), W_embed[:, HC:])
    bea = _lane4(b_embed[:HC])
    beb = _lane4(b_embed[HC:])

    def wbd(l):
        w = Ws[l]
        return (_bd(w[:HC, :HC]), _bd(w[HC:, :HC]),
                _bd(w[:HC, HC:]), _bd(w[HC:, HC:]))

    def layer_consts(l):
        return (_lane4(bs[l][:HC]), _lane4(bs[l][HC:]),
                _lane4(gammas[l][:HC]), _lane4(gammas[l][HC:]),
                _lane4(betas[l][:HC]), _lane4(betas[l][HC:]))

    d0, d1 = _sc_degree(col)
    h_a, h_b, dis, hwa, hwb = _tc_pre(
        xp, d0.reshape(NP, 128), d1.reshape(NP, 128),
        wea, web, bea, beb, wbd(0))
    for l in range(L):
        s0, s1 = _sc_layer(rowL, colL,
                           hwa.reshape(S_ROWS, HC), hwb.reshape(S_ROWS, HC))
        s0 = s0.reshape(NP, 128)
        s1 = s1.reshape(NP, 128)
        if l < L - 1:
            h_a, h_b, hwa, hwb = _tc_layer(
                h_a, h_b, dis, s0, s1, hwa, hwb,
                layer_consts(l), m, wbd(l + 1))
        else:
            out = _tc_final(h_a, h_b, dis, s0, s1, hwa, hwb,
                            layer_consts(l), m, bp,
                            W_o1, b_o1.reshape(1, OUT),
                            W_o2, b_o2.reshape(1, OUT))
    return out
